# baseline (speedup 1.0000x reference)
import functools

import jax
import jax.numpy as jnp
from jax import lax
from jax.experimental import pallas as pl
from jax.experimental.pallas import tpu as pltpu
from jax.experimental.pallas import tpu_sc as plsc

N = 256
LS = 2.0
HS = LS / (N - 1)

B = 2097152
NW = 32
PPW = B // NW
CHUNK = 8192
NSTEP = PPW // CHUNK
LANES = 16


def _quant_body(x0_hbm, x1_hbm, x2_hbm, idx_hbm,
                xa0, xa1, xa2, xb0, xb1, xb2, idxa, idxb, sxa, sxb, soa, sob):
    wid = lax.axis_index("s") * 2 + lax.axis_index("c")
    hs = jnp.float32(HS)
    maxv = jnp.float32(N - 1)
    xv = ((xa0, xa1, xa2), (xb0, xb1, xb2))
    idxv = (idxa, idxb)
    sx = (sxa, sxb)
    so = (soa, sob)

    def quant(v):
        r = (v + 1.0) / hs
        r = jnp.minimum(jnp.maximum(r, 0.0), maxv)
        return r.astype(jnp.int32)

    def start_x(k):
        b = k % 2
        base_pt = wid * PPW + k * CHUNK
        return [
            pltpu.async_copy(xr.at[pl.ds(base_pt, CHUNK)], xv[b][c], sx[b])
            for c, xr in enumerate((x0_hbm, x1_hbm, x2_hbm))
        ]

    hx = {0: start_x(0)}
    ho = {}
    for k in range(NSTEP):
        b = k % 2
        for h in hx[k]:
            h.wait()
        if k + 1 < NSTEP:
            hx[k + 1] = start_x(k + 1)
        if k >= 2:
            ho[k - 2].wait()

        @plsc.parallel_loop(0, CHUNK // LANES, unroll=8)
        def body(j):
            s = pl.ds(j * LANES, LANES)
            i0 = quant(xv[b][0][s])
            i1 = quant(xv[b][1][s])
            i2 = quant(xv[b][2][s])
            idxv[b][s] = i0 * (N * N) + i1 * N + i2

        base_pt = wid * PPW + k * CHUNK
        ho[k] = pltpu.async_copy(idxv[b], idx_hbm.at[pl.ds(base_pt, CHUNK)], so[b])
    for k in (NSTEP - 2, NSTEP - 1):
        ho[k].wait()


def _gather_body(f_hbm, idx_hbm, o_hbm,
                 idxa, idxb, resa, resb, sia, sib, sga, sgb):
    wid = lax.axis_index("s") * 2 + lax.axis_index("c")
    idxv = (idxa, idxb)
    resv = (resa, resb)
    si = (sia, sib)
    sg = (sga, sgb)

    def start_i(k):
        b = k % 2
        base_pt = wid * PPW + k * CHUNK
        return pltpu.async_copy(idx_hbm.at[pl.ds(base_pt, CHUNK)], idxv[b], si[b])

    hi = {0: start_i(0)}
    hg = {}
    for k in range(NSTEP):
        b = k % 2
        hi[k].wait()
        if k + 1 < NSTEP:
            hi[k + 1] = start_i(k + 1)
        if k >= 1:
            hg[k - 1].wait()
            base_prev = wid * PPW + (k - 1) * CHUNK
            pltpu.sync_copy(resv[1 - b], o_hbm.at[pl.ds(base_prev, CHUNK)])
        hg[k] = pltpu.async_copy(f_hbm.at[idxv[b]], resv[b], sg[b])

    hg[NSTEP - 1].wait()
    base_last = wid * PPW + (NSTEP - 1) * CHUNK
    pltpu.sync_copy(resv[(NSTEP - 1) % 2], o_hbm.at[pl.ds(base_last, CHUNK)])


_SC_MESH = dict(
    mesh=plsc.VectorSubcoreMesh(core_axis_name="c", subcore_axis_name="s"),
    compiler_params=pltpu.CompilerParams(needs_layout_passes=False),
)


@jax.jit
def kernel(x, f):
    quant_call = pl.kernel(
        _quant_body,
        out_type=jax.ShapeDtypeStruct((B,), jnp.int32),
        scratch_types=[
            pltpu.VMEM((CHUNK,), jnp.float32),
            pltpu.VMEM((CHUNK,), jnp.float32),
            pltpu.VMEM((CHUNK,), jnp.float32),
            pltpu.VMEM((CHUNK,), jnp.float32),
            pltpu.VMEM((CHUNK,), jnp.float32),
            pltpu.VMEM((CHUNK,), jnp.float32),
            pltpu.VMEM((CHUNK,), jnp.int32),
            pltpu.VMEM((CHUNK,), jnp.int32),
            pltpu.SemaphoreType.DMA,
            pltpu.SemaphoreType.DMA,
            pltpu.SemaphoreType.DMA,
            pltpu.SemaphoreType.DMA,
        ],
        **_SC_MESH,
    )
    gather_call = pl.kernel(
        _gather_body,
        out_type=jax.ShapeDtypeStruct((B,), jnp.float32),
        scratch_types=[
            pltpu.VMEM((CHUNK,), jnp.int32),
            pltpu.VMEM((CHUNK,), jnp.int32),
            pltpu.VMEM((CHUNK,), jnp.float32),
            pltpu.VMEM((CHUNK,), jnp.float32),
            pltpu.SemaphoreType.DMA,
            pltpu.SemaphoreType.DMA,
            pltpu.SemaphoreType.DMA,
            pltpu.SemaphoreType.DMA,
        ],
        **_SC_MESH,
    )
    one = lax.optimization_barrier(jnp.float32(1.0))
    x0 = x[:, 0] * one
    x1 = x[:, 1] * one
    x2 = x[:, 2] * one
    f_lin = f.reshape(N * N * N) * one
    idx = quant_call(x0, x1, x2)
    return gather_call(f_lin, idx)


# speedup vs baseline: 1.0008x; 1.0008x over previous
import functools

import jax
import jax.numpy as jnp
from jax import lax
from jax.experimental import pallas as pl
from jax.experimental.pallas import tpu as pltpu
from jax.experimental.pallas import tpu_sc as plsc

N = 256
LS = 2.0
HS = LS / (N - 1)

B = 2097152
NW = 32
PPW = B // NW
CHUNK = 8192
NSTEP = PPW // CHUNK
LANES = 16


def _quant_body(x0_hbm, x1_hbm, x2_hbm, idx_hbm,
                xa0, xa1, xa2, xb0, xb1, xb2, idxa, idxb, sxa, sxb, soa, sob):
    wid = lax.axis_index("s") * 2 + lax.axis_index("c")
    hs = jnp.float32(HS)
    maxv = jnp.float32(N - 1)
    xv = ((xa0, xa1, xa2), (xb0, xb1, xb2))
    idxv = (idxa, idxb)
    sx = (sxa, sxb)
    so = (soa, sob)

    def quant(v):
        r = (v + 1.0) / hs
        r = jnp.minimum(jnp.maximum(r, 0.0), maxv)
        return r.astype(jnp.int32)

    def start_x(k):
        b = k % 2
        base_pt = wid * PPW + k * CHUNK
        return [
            pltpu.async_copy(xr.at[pl.ds(base_pt, CHUNK)], xv[b][c], sx[b])
            for c, xr in enumerate((x0_hbm, x1_hbm, x2_hbm))
        ]

    hx = {0: start_x(0)}
    ho = {}
    for k in range(NSTEP):
        b = k % 2
        for h in hx[k]:
            h.wait()
        if k + 1 < NSTEP:
            hx[k + 1] = start_x(k + 1)
        if k >= 2:
            ho[k - 2].wait()

        @plsc.parallel_loop(0, CHUNK // LANES, unroll=8)
        def body(j):
            s = pl.ds(j * LANES, LANES)
            i0 = quant(xv[b][0][s])
            i1 = quant(xv[b][1][s])
            i2 = quant(xv[b][2][s])
            idxv[b][s] = i0 * (N * N) + i1 * N + i2

        base_pt = wid * PPW + k * CHUNK
        ho[k] = pltpu.async_copy(idxv[b], idx_hbm.at[pl.ds(base_pt, CHUNK)], so[b])
    for k in (NSTEP - 2, NSTEP - 1):
        ho[k].wait()


def _gather_body(f_hbm, idx_hbm, o_hbm,
                 idxa, idxb, resa, resb, sia, sib, sga, sgb):
    wid = lax.axis_index("s") * 2 + lax.axis_index("c")
    idxv = (idxa, idxb)
    resv = (resa, resb)
    si = (sia, sib)
    sg = (sga, sgb)

    def start_i(k):
        b = k % 2
        base_pt = wid * PPW + k * CHUNK
        return pltpu.async_copy(idx_hbm.at[pl.ds(base_pt, CHUNK)], idxv[b], si[b])

    hi = {0: start_i(0)}
    hg = {}
    for k in range(NSTEP):
        b = k % 2
        hi[k].wait()
        if k >= 1:
            hg[k - 1].wait()
            base_prev = wid * PPW + (k - 1) * CHUNK
            pltpu.sync_copy(resv[1 - b], o_hbm.at[pl.ds(base_prev, CHUNK)])
        if k + 1 < NSTEP:
            hi[k + 1] = start_i(k + 1)
        hg[k] = pltpu.async_copy(f_hbm.at[idxv[b]], resv[b], sg[b])

    hg[NSTEP - 1].wait()
    base_last = wid * PPW + (NSTEP - 1) * CHUNK
    pltpu.sync_copy(resv[(NSTEP - 1) % 2], o_hbm.at[pl.ds(base_last, CHUNK)])


_SC_MESH = dict(
    mesh=plsc.VectorSubcoreMesh(core_axis_name="c", subcore_axis_name="s"),
    compiler_params=pltpu.CompilerParams(needs_layout_passes=False),
)


@jax.jit
def kernel(x, f):
    quant_call = pl.kernel(
        _quant_body,
        out_type=jax.ShapeDtypeStruct((B,), jnp.int32),
        scratch_types=[
            pltpu.VMEM((CHUNK,), jnp.float32),
            pltpu.VMEM((CHUNK,), jnp.float32),
            pltpu.VMEM((CHUNK,), jnp.float32),
            pltpu.VMEM((CHUNK,), jnp.float32),
            pltpu.VMEM((CHUNK,), jnp.float32),
            pltpu.VMEM((CHUNK,), jnp.float32),
            pltpu.VMEM((CHUNK,), jnp.int32),
            pltpu.VMEM((CHUNK,), jnp.int32),
            pltpu.SemaphoreType.DMA,
            pltpu.SemaphoreType.DMA,
            pltpu.SemaphoreType.DMA,
            pltpu.SemaphoreType.DMA,
        ],
        **_SC_MESH,
    )
    gather_call = pl.kernel(
        _gather_body,
        out_type=jax.ShapeDtypeStruct((B,), jnp.float32),
        scratch_types=[
            pltpu.VMEM((CHUNK,), jnp.int32),
            pltpu.VMEM((CHUNK,), jnp.int32),
            pltpu.VMEM((CHUNK,), jnp.float32),
            pltpu.VMEM((CHUNK,), jnp.float32),
            pltpu.SemaphoreType.DMA,
            pltpu.SemaphoreType.DMA,
            pltpu.SemaphoreType.DMA,
            pltpu.SemaphoreType.DMA,
        ],
        **_SC_MESH,
    )
    one = lax.optimization_barrier(jnp.float32(1.0))
    x0 = x[:, 0] * one
    x1 = x[:, 1] * one
    x2 = x[:, 2] * one
    f_lin = f.reshape(N * N * N) * one
    idx = quant_call(x0, x1, x2)
    return gather_call(f_lin, idx)


# speedup vs baseline: 1.2818x; 1.2807x over previous
import functools

import jax
import jax.numpy as jnp
from jax import lax
from jax.experimental import pallas as pl
from jax.experimental.pallas import tpu as pltpu
from jax.experimental.pallas import tpu_sc as plsc

N = 256
LS = 2.0
HS = LS / (N - 1)

B = 2097152
NW = 32
PPW = B // NW
CHUNK = 8192
NSTEP = PPW // CHUNK
LANES = 16


def _sc_body(
    x0_hbm, x1_hbm, x2_hbm, f_hbm, o_hbm,
    xa0, xa1, xa2, xb0, xb1, xb2,
    idx0, idx1, idx2, res0, res1, res2,
    sxa, sxb, sg0, sg1, sg2,
):
    wid = lax.axis_index("s") * 2 + lax.axis_index("c")
    hs = jnp.float32(HS)
    maxv = jnp.float32(N - 1)
    xv = ((xa0, xa1, xa2), (xb0, xb1, xb2))
    idxv = (idx0, idx1, idx2)
    resv = (res0, res1, res2)
    sg = (sg0, sg1, sg2)
    sx = (sxa, sxb)

    def quant(v):
        r = (v + 1.0) / hs
        r = jnp.minimum(jnp.maximum(r, 0.0), maxv)
        return r.astype(jnp.int32)

    def start_x(k):
        b = k % 2
        base_pt = wid * PPW + k * CHUNK
        return [
            pltpu.async_copy(xr.at[pl.ds(base_pt, CHUNK)], xv[b][c], sx[b])
            for c, xr in enumerate((x0_hbm, x1_hbm, x2_hbm))
        ]

    def drain(k):
        hg[k].wait()
        base = wid * PPW + k * CHUNK
        pltpu.sync_copy(resv[k % 3], o_hbm.at[pl.ds(base, CHUNK)])

    hx = {0: start_x(0)}
    hg = {}
    for k in range(NSTEP):
        b = k % 2
        for h in hx[k]:
            h.wait()
        if k + 1 < NSTEP:
            hx[k + 1] = start_x(k + 1)
        if k >= 2:
            drain(k - 2)

        @plsc.parallel_loop(0, CHUNK // LANES, unroll=8)
        def body(j):
            s = pl.ds(j * LANES, LANES)
            i0 = quant(xv[b][0][s])
            i1 = quant(xv[b][1][s])
            i2 = quant(xv[b][2][s])
            idxv[k % 3][s] = i0 * (N * N) + i1 * N + i2

        hg[k] = pltpu.async_copy(f_hbm.at[idxv[k % 3]], resv[k % 3], sg[k % 3])

    drain(NSTEP - 2)
    drain(NSTEP - 1)


@jax.jit
def kernel(x, f):
    sc_call = pl.kernel(
        _sc_body,
        out_type=jax.ShapeDtypeStruct((B,), jnp.float32),
        mesh=plsc.VectorSubcoreMesh(core_axis_name="c", subcore_axis_name="s"),
        compiler_params=pltpu.CompilerParams(needs_layout_passes=False),
        scratch_types=[
            pltpu.VMEM((CHUNK,), jnp.float32),
            pltpu.VMEM((CHUNK,), jnp.float32),
            pltpu.VMEM((CHUNK,), jnp.float32),
            pltpu.VMEM((CHUNK,), jnp.float32),
            pltpu.VMEM((CHUNK,), jnp.float32),
            pltpu.VMEM((CHUNK,), jnp.float32),
            pltpu.VMEM((CHUNK,), jnp.int32),
            pltpu.VMEM((CHUNK,), jnp.int32),
            pltpu.VMEM((CHUNK,), jnp.int32),
            pltpu.VMEM((CHUNK,), jnp.float32),
            pltpu.VMEM((CHUNK,), jnp.float32),
            pltpu.VMEM((CHUNK,), jnp.float32),
            pltpu.SemaphoreType.DMA,
            pltpu.SemaphoreType.DMA,
            pltpu.SemaphoreType.DMA,
            pltpu.SemaphoreType.DMA,
            pltpu.SemaphoreType.DMA,
        ],
    )
    one = lax.optimization_barrier(jnp.float32(1.0))
    x0 = x[:, 0] * one
    x1 = x[:, 1] * one
    x2 = x[:, 2] * one
    f_lin = f.reshape(N * N * N)
    return sc_call(x0, x1, x2, f_lin)


# speedup vs baseline: 1.2879x; 1.0047x over previous
import functools

import jax
import jax.numpy as jnp
from jax import lax
from jax.experimental import pallas as pl
from jax.experimental.pallas import tpu as pltpu
from jax.experimental.pallas import tpu_sc as plsc

N = 256
LS = 2.0
HS = LS / (N - 1)

B = 2097152
NW = 32
PPW = B // NW
CHUNK = 8192
NSTEP = PPW // CHUNK
LANES = 16


def _sc_body(
    x0_hbm, x1_hbm, x2_hbm, f_hbm, o_hbm,
    xa0, xa1, xa2, xb0, xb1, xb2,
    idx0, idx1, idx2, idx3, res0, res1, res2, res3,
    sxa, sxb, sg0, sg1, sg2, sg3,
):
    wid = lax.axis_index("s") * 2 + lax.axis_index("c")
    hs = jnp.float32(HS)
    maxv = jnp.float32(N - 1)
    xv = ((xa0, xa1, xa2), (xb0, xb1, xb2))
    idxv = (idx0, idx1, idx2, idx3)
    resv = (res0, res1, res2, res3)
    sg = (sg0, sg1, sg2, sg3)
    sx = (sxa, sxb)

    def quant(v):
        r = (v + 1.0) / hs
        r = jnp.minimum(jnp.maximum(r, 0.0), maxv)
        return r.astype(jnp.int32)

    def start_x(k):
        b = k % 2
        base_pt = wid * PPW + k * CHUNK
        return [
            pltpu.async_copy(xr.at[pl.ds(base_pt, CHUNK)], xv[b][c], sx[b])
            for c, xr in enumerate((x0_hbm, x1_hbm, x2_hbm))
        ]

    def drain(k):
        hg[k].wait()
        base = wid * PPW + k * CHUNK
        pltpu.sync_copy(resv[k % 4], o_hbm.at[pl.ds(base, CHUNK)])

    hx = {0: start_x(0)}
    hg = {}
    for k in range(NSTEP):
        b = k % 2
        for h in hx[k]:
            h.wait()
        if k + 1 < NSTEP:
            hx[k + 1] = start_x(k + 1)
        if k >= 3:
            drain(k - 3)

        @plsc.parallel_loop(0, CHUNK // LANES, unroll=8)
        def body(j):
            s = pl.ds(j * LANES, LANES)
            i0 = quant(xv[b][0][s])
            i1 = quant(xv[b][1][s])
            i2 = quant(xv[b][2][s])
            idxv[k % 4][s] = i0 * (N * N) + i1 * N + i2

        hg[k] = pltpu.async_copy(f_hbm.at[idxv[k % 4]], resv[k % 4], sg[k % 4])

    drain(NSTEP - 3)
    drain(NSTEP - 2)
    drain(NSTEP - 1)


@jax.jit
def kernel(x, f):
    sc_call = pl.kernel(
        _sc_body,
        out_type=jax.ShapeDtypeStruct((B,), jnp.float32),
        mesh=plsc.VectorSubcoreMesh(core_axis_name="c", subcore_axis_name="s"),
        compiler_params=pltpu.CompilerParams(needs_layout_passes=False),
        scratch_types=[
            pltpu.VMEM((CHUNK,), jnp.float32),
            pltpu.VMEM((CHUNK,), jnp.float32),
            pltpu.VMEM((CHUNK,), jnp.float32),
            pltpu.VMEM((CHUNK,), jnp.float32),
            pltpu.VMEM((CHUNK,), jnp.float32),
            pltpu.VMEM((CHUNK,), jnp.float32),
            pltpu.VMEM((CHUNK,), jnp.int32),
            pltpu.VMEM((CHUNK,), jnp.int32),
            pltpu.VMEM((CHUNK,), jnp.int32),
            pltpu.VMEM((CHUNK,), jnp.int32),
            pltpu.VMEM((CHUNK,), jnp.float32),
            pltpu.VMEM((CHUNK,), jnp.float32),
            pltpu.VMEM((CHUNK,), jnp.float32),
            pltpu.VMEM((CHUNK,), jnp.float32),
            pltpu.SemaphoreType.DMA,
            pltpu.SemaphoreType.DMA,
            pltpu.SemaphoreType.DMA,
            pltpu.SemaphoreType.DMA,
            pltpu.SemaphoreType.DMA,
            pltpu.SemaphoreType.DMA,
        ],
    )
    one = lax.optimization_barrier(jnp.float32(1.0))
    x0 = x[:, 0] * one
    x1 = x[:, 1] * one
    x2 = x[:, 2] * one
    f_lin = f.reshape(N * N * N)
    return sc_call(x0, x1, x2, f_lin)
